# baseline (device time: 232152 ns/iter reference)
import functools

import jax
import jax.numpy as jnp
from jax import lax
from jax.experimental import pallas as pl
from jax.experimental.pallas import tpu as pltpu

N_DEV = 16
N_FLOWS = 8
DIRS = (1, 1, 1, 1, -1, -1, -1, -1)


def kernel(x, w_mat, scale_x, scale_w):
    m_rows = x.shape[0]
    n_cols = w_mat.shape[1]
    ch = m_rows // N_DEV
    colw = n_cols // N_FLOWS

    x_bf = x.astype(jnp.bfloat16)
    w_bf = (w_mat.astype(jnp.float32)
            * (scale_x[0] * scale_w[0])).astype(jnp.bfloat16)

    def body(x_ref, w_ref, out_hbm, comm, outstage, store_sems,
             rs_ssem, rs_rsem, ag_ssem, ag_rsem):
        me = lax.axis_index("i")
        left = lax.rem(me - 1 + N_DEV, N_DEV)
        right = lax.rem(me + 1, N_DEV)

        def pmod(v):
            return lax.rem(v + 2 * N_DEV, N_DEV)

        barrier = pltpu.get_barrier_semaphore()
        for nbr in (left, right):
            pl.semaphore_signal(barrier, inc=1, device_id=(nbr,),
                                device_id_type=pl.DeviceIdType.MESH)
        pl.semaphore_wait(barrier, 2)

        def tile(c, f):
            return lax.dot_general(
                x_ref[pl.ds(c * ch, ch), :],
                w_ref[:, pl.ds(f * colw, colw)],
                dimension_numbers=(((1,), (0,)), ((), ())),
                preferred_element_type=jnp.float32,
            )

        def rs_rdma(f, s):
            return pltpu.make_async_remote_copy(
                src_ref=comm.at[f, s],
                dst_ref=comm.at[f, s + 1],
                send_sem=rs_ssem.at[f, s],
                recv_sem=rs_rsem.at[f, s],
                device_id=(right if DIRS[f] > 0 else left,),
                device_id_type=pl.DeviceIdType.MESH,
            )

        def ag_rdma(f, t):
            return pltpu.make_async_remote_copy(
                src_ref=comm.at[f, N_DEV - 1 if t == 0 else t - 1],
                dst_ref=comm.at[f, t],
                send_sem=ag_ssem.at[f, t],
                recv_sem=ag_rsem.at[f, t],
                device_id=(right if DIRS[f] > 0 else left,),
                device_id_type=pl.DeviceIdType.MESH,
            )

        all_sends = []
        store_hist = [[] for _ in range(N_FLOWS)]

        def emit_store(f, k, src_slot, chunk):
            if k >= 2:
                store_hist[f][k - 2].wait()
            outstage[f, k % 2, :, :] = comm[f, src_slot, :, :].astype(
                jnp.float32)
            st = pltpu.make_async_copy(
                outstage.at[f, k % 2],
                out_hbm.at[pl.ds(chunk * ch, ch), pl.ds(f * colw, colw)],
                store_sems.at[f, k],
            )
            st.start()
            store_hist[f].append(st)

        inflight = []
        for f in range(N_FLOWS):
            comm[f, 0, :, :] = tile(me, f).astype(jnp.bfloat16)
            rdma = rs_rdma(f, 0)
            rdma.start()
            inflight.append(rdma)
            all_sends.append(rdma)

        for s in range(N_DEV - 1):
            for f in range(N_FLOWS):
                inflight[f].wait_recv()
                acc = (comm[f, s + 1, :, :].astype(jnp.float32)
                       + tile(pmod(me - DIRS[f] * (s + 1)), f))
                comm[f, s + 1, :, :] = acc.astype(jnp.bfloat16)
                if s + 1 < N_DEV - 1:
                    rdma = rs_rdma(f, s + 1)
                    rdma.start()
                    inflight[f] = rdma
                    all_sends.append(rdma)

        for f in range(N_FLOWS):
            rdma = ag_rdma(f, 0)
            rdma.start()
            inflight[f] = rdma
            all_sends.append(rdma)
            emit_store(f, 0, N_DEV - 1, pmod(me + DIRS[f]))

        for t in range(N_DEV - 1):
            for f in range(N_FLOWS):
                inflight[f].wait_recv()
                if t + 1 < N_DEV - 1:
                    rdma = ag_rdma(f, t + 1)
                    rdma.start()
                    inflight[f] = rdma
                    all_sends.append(rdma)
                emit_store(f, t + 1, t, pmod(me - DIRS[f] * t))

        for rdma in all_sends:
            rdma.wait_send()
        for f in range(N_FLOWS):
            for st in store_hist[f][-2:]:
                st.wait()

        @functools.partial(pl.run_scoped,
                           exit_sem=pltpu.SemaphoreType.REGULAR)
        def _(exit_sem):
            for nbr in (left, right):
                pl.semaphore_signal(exit_sem, inc=1, device_id=(nbr,),
                                    device_id_type=pl.DeviceIdType.MESH)
            pl.semaphore_wait(exit_sem, 2)

    return pl.pallas_call(
        body,
        out_shape=jax.ShapeDtypeStruct((m_rows, n_cols), jnp.float32),
        in_specs=[
            pl.BlockSpec(memory_space=pltpu.MemorySpace.VMEM),
            pl.BlockSpec(memory_space=pltpu.MemorySpace.VMEM),
        ],
        out_specs=pl.BlockSpec(memory_space=pltpu.MemorySpace.HBM),
        scratch_shapes=[
            pltpu.VMEM((N_FLOWS, N_DEV, ch, colw), jnp.bfloat16),
            pltpu.VMEM((N_FLOWS, 2, ch, colw), jnp.float32),
            pltpu.SemaphoreType.DMA((N_FLOWS, N_DEV)),
            pltpu.SemaphoreType.DMA((N_FLOWS, N_DEV - 1)),
            pltpu.SemaphoreType.DMA((N_FLOWS, N_DEV - 1)),
            pltpu.SemaphoreType.DMA((N_FLOWS, N_DEV - 1)),
            pltpu.SemaphoreType.DMA((N_FLOWS, N_DEV - 1)),
        ],
        compiler_params=pltpu.CompilerParams(
            collective_id=0,
            vmem_limit_bytes=56 * 1024 * 1024,
        ),
    )(x_bf, w_bf)


# device time: 216394 ns/iter; 1.0728x vs baseline; 1.0728x over previous
import functools

import jax
import jax.numpy as jnp
from jax import lax
from jax.experimental import pallas as pl
from jax.experimental.pallas import tpu as pltpu

N_DEV = 16
N_FLOWS = 4
DIRS = (1, 1, -1, -1)
FLOW_ORDER = (0, 2, 1, 3)


def kernel(x, w_mat, scale_x, scale_w):
    m_rows = x.shape[0]
    n_cols = w_mat.shape[1]
    ch = m_rows // N_DEV
    colw = n_cols // N_FLOWS

    x_bf = x.astype(jnp.bfloat16)
    w_bf = (w_mat.astype(jnp.float32)
            * (scale_x[0] * scale_w[0])).astype(jnp.bfloat16)

    def body(x_ref, w_ref, out_hbm, comm, outstage, store_sems,
             rs_ssem, rs_rsem, ag_ssem, ag_rsem):
        me = lax.axis_index("i")
        left = lax.rem(me - 1 + N_DEV, N_DEV)
        right = lax.rem(me + 1, N_DEV)

        def pmod(v):
            return lax.rem(v + 2 * N_DEV, N_DEV)

        barrier = pltpu.get_barrier_semaphore()
        for nbr in (left, right):
            pl.semaphore_signal(barrier, inc=1, device_id=(nbr,),
                                device_id_type=pl.DeviceIdType.MESH)
        pl.semaphore_wait(barrier, 2)

        def tile(c, f):
            return lax.dot_general(
                x_ref[pl.ds(c * ch, ch), :],
                w_ref[:, pl.ds(f * colw, colw)],
                dimension_numbers=(((1,), (0,)), ((), ())),
                preferred_element_type=jnp.float32,
            )

        def rs_rdma(f, s):
            return pltpu.make_async_remote_copy(
                src_ref=comm.at[f, s],
                dst_ref=comm.at[f, s + 1],
                send_sem=rs_ssem.at[f, s],
                recv_sem=rs_rsem.at[f, s],
                device_id=(right if DIRS[f] > 0 else left,),
                device_id_type=pl.DeviceIdType.MESH,
            )

        def ag_rdma(f, t):
            return pltpu.make_async_remote_copy(
                src_ref=comm.at[f, N_DEV - 1 if t == 0 else t - 1],
                dst_ref=comm.at[f, t],
                send_sem=ag_ssem.at[f, t],
                recv_sem=ag_rsem.at[f, t],
                device_id=(right if DIRS[f] > 0 else left,),
                device_id_type=pl.DeviceIdType.MESH,
            )

        all_sends = []
        store_hist = [[] for _ in range(N_FLOWS)]

        def emit_store(f, k, src_slot, chunk):
            if k >= 2:
                store_hist[f][k - 2].wait()
            outstage[f, k % 2, :, :] = comm[f, src_slot, :, :].astype(
                jnp.float32)
            st = pltpu.make_async_copy(
                outstage.at[f, k % 2],
                out_hbm.at[pl.ds(chunk * ch, ch), pl.ds(f * colw, colw)],
                store_sems.at[f, k],
            )
            st.start()
            store_hist[f].append(st)

        inflight = []
        for f in range(N_FLOWS):
            comm[f, 0, :, :] = tile(me, f).astype(jnp.bfloat16)
            rdma = rs_rdma(f, 0)
            rdma.start()
            inflight.append(rdma)
            all_sends.append(rdma)

        for s in range(N_DEV - 1):
            for f in FLOW_ORDER:
                inflight[f].wait_recv()
                acc = (comm[f, s + 1, :, :].astype(jnp.float32)
                       + tile(pmod(me - DIRS[f] * (s + 1)), f))
                comm[f, s + 1, :, :] = acc.astype(jnp.bfloat16)
                if s + 1 < N_DEV - 1:
                    rdma = rs_rdma(f, s + 1)
                    rdma.start()
                    inflight[f] = rdma
                    all_sends.append(rdma)

        for f in range(N_FLOWS):
            rdma = ag_rdma(f, 0)
            rdma.start()
            inflight[f] = rdma
            all_sends.append(rdma)
            emit_store(f, 0, N_DEV - 1, pmod(me + DIRS[f]))

        for t in range(N_DEV - 1):
            for f in FLOW_ORDER:
                inflight[f].wait_recv()
                if t + 1 < N_DEV - 1:
                    rdma = ag_rdma(f, t + 1)
                    rdma.start()
                    inflight[f] = rdma
                    all_sends.append(rdma)
                emit_store(f, t + 1, t, pmod(me - DIRS[f] * t))

        for rdma in all_sends:
            rdma.wait_send()
        for f in range(N_FLOWS):
            for st in store_hist[f][-2:]:
                st.wait()

        @functools.partial(pl.run_scoped,
                           exit_sem=pltpu.SemaphoreType.REGULAR)
        def _(exit_sem):
            for nbr in (left, right):
                pl.semaphore_signal(exit_sem, inc=1, device_id=(nbr,),
                                    device_id_type=pl.DeviceIdType.MESH)
            pl.semaphore_wait(exit_sem, 2)

    return pl.pallas_call(
        body,
        out_shape=jax.ShapeDtypeStruct((m_rows, n_cols), jnp.float32),
        in_specs=[
            pl.BlockSpec(memory_space=pltpu.MemorySpace.VMEM),
            pl.BlockSpec(memory_space=pltpu.MemorySpace.VMEM),
        ],
        out_specs=pl.BlockSpec(memory_space=pltpu.MemorySpace.HBM),
        scratch_shapes=[
            pltpu.VMEM((N_FLOWS, N_DEV, ch, colw), jnp.bfloat16),
            pltpu.VMEM((N_FLOWS, 2, ch, colw), jnp.float32),
            pltpu.SemaphoreType.DMA((N_FLOWS, N_DEV)),
            pltpu.SemaphoreType.DMA((N_FLOWS, N_DEV - 1)),
            pltpu.SemaphoreType.DMA((N_FLOWS, N_DEV - 1)),
            pltpu.SemaphoreType.DMA((N_FLOWS, N_DEV - 1)),
            pltpu.SemaphoreType.DMA((N_FLOWS, N_DEV - 1)),
        ],
        compiler_params=pltpu.CompilerParams(
            collective_id=0,
            vmem_limit_bytes=56 * 1024 * 1024,
        ),
    )(x_bf, w_bf)
